# baseline (speedup 1.0000x reference)
import functools

import jax
import jax.numpy as jnp
from jax import lax
from jax.experimental import pallas as pl
from jax.experimental.pallas import tpu as pltpu
from jax.experimental.pallas import tpu_sc as plsc

_NC = 2
_NS = 16
_NW = _NC * _NS
_CHUNK = 128


@functools.lru_cache(maxsize=None)
def _make_gather(B, D):
    b_per_w = B // _NW
    nchunks = b_per_w // _CHUNK
    mesh = plsc.VectorSubcoreMesh(
        core_axis_name="c", subcore_axis_name="s",
        num_cores=_NC, num_subcores=_NS)

    @functools.partial(
        pl.kernel,
        out_type=(jax.ShapeDtypeStruct((B, D), jnp.float32),
                  jax.ShapeDtypeStruct((B, D), jnp.float32)),
        mesh=mesh,
        scratch_types=[
            pltpu.VMEM((b_per_w,), jnp.int32),
            pltpu.VMEM((b_per_w,), jnp.int32),
            pltpu.VMEM((_CHUNK, D), jnp.float32),
            pltpu.VMEM((_CHUNK, D), jnp.float32),
            pltpu.SemaphoreType.DMA,
        ],
        compiler_params=pltpu.CompilerParams(use_tc_tiling_on_sc=False),
    )
    def gather(a_hbm, b_hbm, ea_hbm, eb_hbm, xa_hbm, xb_hbm,
               ia_v, ib_v, ra_v, rb_v, sem):
        wid = lax.axis_index("s") * _NC + lax.axis_index("c")
        base = wid * b_per_w
        pltpu.sync_copy(a_hbm.at[pl.ds(base, b_per_w)], ia_v)
        pltpu.sync_copy(b_hbm.at[pl.ds(base, b_per_w)], ib_v)
        for t in range(nchunks):
            ca = pltpu.async_copy(
                ea_hbm.at[ia_v.at[pl.ds(t * _CHUNK, _CHUNK)]], ra_v, sem)
            cb = pltpu.async_copy(
                eb_hbm.at[ib_v.at[pl.ds(t * _CHUNK, _CHUNK)]], rb_v, sem)
            ca.wait()
            cb.wait()
            pltpu.sync_copy(ra_v, xa_hbm.at[pl.ds(base + t * _CHUNK, _CHUNK)])
            pltpu.sync_copy(rb_v, xb_hbm.at[pl.ds(base + t * _CHUNK, _CHUNK)])

    return gather


def _mlp_body(xa_ref, xb_ref, w1_ref, b1_ref, w2_ref, b2_ref, out_ref):
    x = xa_ref[...] + xb_ref[...]
    h = lax.dot_general(x, w1_ref[...], (((1,), (1,)), ((), ())),
                        preferred_element_type=jnp.float32)
    h = jnp.maximum(h + b1_ref[...], 0.0)
    out = lax.dot_general(h, w2_ref[...], (((1,), (1,)), ((), ())),
                          preferred_element_type=jnp.float32)
    out_ref[...] = out + b2_ref[...]


@functools.lru_cache(maxsize=None)
def _make_mlp(B, D, H, P, BT=512, PT=2048):
    nb = B // BT
    npt = pl.cdiv(P, PT)
    grid = (npt, nb)
    return pl.pallas_call(
        _mlp_body,
        grid=grid,
        in_specs=[
            pl.BlockSpec((BT, D), lambda j, i: (i, 0)),
            pl.BlockSpec((BT, D), lambda j, i: (i, 0)),
            pl.BlockSpec((H, D), lambda j, i: (0, 0)),
            pl.BlockSpec((1, H), lambda j, i: (0, 0)),
            pl.BlockSpec((PT, H), lambda j, i: (j, 0)),
            pl.BlockSpec((1, PT), lambda j, i: (0, j)),
        ],
        out_specs=pl.BlockSpec((BT, PT), lambda j, i: (i, j)),
        out_shape=jax.ShapeDtypeStruct((B, P), jnp.float32),
        compiler_params=pltpu.CompilerParams(
            dimension_semantics=("arbitrary", "arbitrary")),
    )


def kernel(a, b, emb_a, emb_b, W1, b1, W2, b2):
    B = a.shape[0]
    P, D = emb_a.shape
    H = W1.shape[0]
    xa, xb = _make_gather(B, D)(a, b, emb_a, emb_b)
    mlp = _make_mlp(B, D, H, P)
    return mlp(xa, xb, W1, b1.reshape(1, H), W2, b2.reshape(1, P))
